# baseline (speedup 1.0000x reference)
import functools
import jax
import jax.numpy as jnp
from jax import lax
from jax.experimental import pallas as pl
from jax.experimental.pallas import tpu as pltpu
from jax.experimental.pallas import tpu_sc as plsc

_DIM = 64
_SCALE = 8.0

_NC = 2
_NS = 16
_NW = _NC * _NS
_CH = 128


def _make_kernel(n_chunks):
    b_per_w = n_chunks * _CH
    mesh = plsc.VectorSubcoreMesh(core_axis_name="c", subcore_axis_name="s")

    @functools.partial(
        pl.kernel,
        mesh=mesh,
        out_type=jax.ShapeDtypeStruct((_NW * b_per_w, _DIM), jnp.float32),
        scratch_types=[
            pltpu.VMEM((n_chunks, _CH), jnp.int32),
            pltpu.VMEM((_CH, _DIM), jnp.float32),
            pltpu.SemaphoreType.DMA,
        ],
        compiler_params=pltpu.CompilerParams(use_tc_tiling_on_sc=False),
    )
    def k(idx_hbm, table_hbm, out_hbm, idx_v, rows_v, sem):
        wid = lax.axis_index("s") * _NC + lax.axis_index("c")
        base = wid * b_per_w
        pltpu.sync_copy(idx_hbm.at[wid], idx_v)

        def chunk_body(c, carry):
            pltpu.async_copy(table_hbm.at[idx_v.at[c]], rows_v, sem).wait()

            def row_body(r, rcarry):
                for j in range(_DIM // 16):
                    sl = pl.ds(j * 16, 16)
                    rows_v[r, sl] = rows_v[r, sl] * _SCALE
                return rcarry

            lax.fori_loop(0, _CH, row_body, 0, unroll=4)
            pltpu.sync_copy(rows_v, out_hbm.at[pl.ds(base + c * _CH, _CH)])
            return carry

        lax.fori_loop(0, n_chunks, chunk_body, 0)

    return k


@jax.jit
def kernel(x, table):
    bsz, seq = x.shape
    total = bsz * seq
    n_chunks = total // (_NW * _CH)
    idx = x.reshape(_NW, n_chunks, _CH).astype(jnp.int32)
    out = _make_kernel(n_chunks)(idx, table)
    return out.reshape(bsz, seq, _DIM)


# speedup vs baseline: 1.1425x; 1.1425x over previous
import functools
import jax
import jax.numpy as jnp
from jax import lax
from jax.experimental import pallas as pl
from jax.experimental.pallas import tpu as pltpu
from jax.experimental.pallas import tpu_sc as plsc

_DIM = 64
_SCALE = 8.0

_NC = 2
_NS = 16
_NW = _NC * _NS
_CH = 128
_NB = 4
_K = 2


def _make_kernel(n_chunks):
    assert n_chunks % _NB == 0 and n_chunks // _NB >= 3
    n_groups = n_chunks // _NB
    b_per_w = n_chunks * _CH
    mesh = plsc.VectorSubcoreMesh(core_axis_name="c", subcore_axis_name="s")

    @functools.partial(
        pl.kernel,
        mesh=mesh,
        out_type=jax.ShapeDtypeStruct((_NW * b_per_w, _DIM), jnp.float32),
        scratch_types=[
            pltpu.VMEM((n_chunks, _CH), jnp.int32),
            pltpu.VMEM((_NB, _CH, _DIM), jnp.float32),
            [pltpu.SemaphoreType.DMA] * _NB,
            [pltpu.SemaphoreType.DMA] * _NB,
        ],
        compiler_params=pltpu.CompilerParams(use_tc_tiling_on_sc=False),
    )
    def k(idx_hbm, table_hbm, out_hbm, idx_v, rows_v, sg, sw):
        wid = lax.axis_index("s") * _NC + lax.axis_index("c")
        base = wid * b_per_w
        pltpu.sync_copy(idx_hbm.at[wid], idx_v)

        def gather(c, b):
            return pltpu.make_async_copy(
                table_hbm.at[idx_v.at[c]], rows_v.at[b], sg[b])

        def write(c, b):
            return pltpu.make_async_copy(
                rows_v.at[b], out_hbm.at[pl.ds(base + c * _CH, _CH)], sw[b])

        def scale(b):
            def row_body(r, rc):
                for j in range(_DIM // 16):
                    sl = pl.ds(j * 16, 16)
                    rows_v[b, r, sl] = rows_v[b, r, sl] * _SCALE
                return rc
            lax.fori_loop(0, _CH, row_body, 0, unroll=4)

        def step(c, b, start_next=True, wait_prev=True):
            gather(c, b).wait()
            scale(b)
            write(c, b).start()
            if start_next:
                bn = (b + _K) % _NB
                if wait_prev:
                    write(c + _K - _NB, bn).wait()
                gather(c + _K, bn).start()

        for b in range(_K):
            gather(b, b).start()
        for b in range(_NB):
            step(b, b, start_next=True, wait_prev=(b + _K >= _NB))

        def group_body(g, carry):
            c0 = g * _NB
            for b in range(_NB):
                step(c0 + b, b)
            return carry
        lax.fori_loop(1, n_groups - 1, group_body, 0)

        c0 = (n_groups - 1) * _NB
        for b in range(_NB):
            step(c0 + b, b, start_next=(b + _K < _NB))

        for b in range(_NB):
            write(c0 + b, b).wait()

    return k


@jax.jit
def kernel(x, table):
    bsz, seq = x.shape
    total = bsz * seq
    n_chunks = total // (_NW * _CH)
    idx = x.reshape(_NW, n_chunks, _CH).astype(jnp.int32)
    out = _make_kernel(n_chunks)(idx, table)
    return out.reshape(bsz, seq, _DIM)
